# Initial kernel scaffold; baseline (speedup 1.0000x reference)
import jax
import jax.numpy as jnp
from jax.experimental import pallas as pl


def kernel(x, edge_index, W1, b1, W2, b2):
    raise NotImplementedError("write your pallas kernel here")


# speedup vs baseline: 13.3015x; 13.3015x over previous
import functools

import jax
import jax.numpy as jnp
from jax import lax
from jax.experimental import pallas as pl
from jax.experimental.pallas import tpu as pltpu
from jax.experimental.pallas import tpu_sc as plsc

N = 10000
E = 320000
D = 128

NW = 32
EPW = E // NW
CB = 128
NCHUNK = 80
EPW_P = NCHUNK * CB
PAD = EPW_P - EPW
ACC_ROWS = N + 112
ZROWS = ACC_ROWS // 16

_mesh = plsc.VectorSubcoreMesh(core_axis_name="c", subcore_axis_name="s")


def _make_sc_pass(dwidth, use_table):
    def body(*refs):
        if use_table:
            srcp, dstp, table, zrows, out, isrc, idst, rows, sem, acc = refs
        else:
            dstp, zrows, ones, out, idst, rows, sem, acc = refs
        c = lax.axis_index("c")
        s = lax.axis_index("s")
        wid = s * 2 + c

        pltpu.sync_copy(zrows, rows.at[0])
        if not use_table:
            pltpu.sync_copy(ones, rows.at[1])

        base = s * ZROWS
        for off, sz in ((0, 128), (128, 128), (256, 128), (384, 128), (512, 120)):
            pltpu.sync_copy(rows.at[0, pl.ds(0, sz)],
                            acc.at[pl.ds(base + off, sz)])

        pltpu.sync_copy(dstp.at[pl.ds(wid * NCHUNK, NCHUNK)], idst)
        if use_table:
            pltpu.sync_copy(srcp.at[pl.ds(wid * NCHUNK, NCHUNK)], isrc)
        plsc.subcore_barrier()

        if use_table:
            def ebody(j, _):
                pltpu.async_copy(table.at[isrc.at[j]], rows.at[0], sem).wait()
                pltpu.sync_copy(rows.at[0], acc.at[idst.at[j]], add=True)
                return 0
        else:
            def ebody(j, _):
                pltpu.sync_copy(rows.at[1], acc.at[idst.at[j]], add=True)
                return 0
        lax.fori_loop(0, NCHUNK, ebody, 0)
        plsc.subcore_barrier()

        for off, sz in ((0, 128), (128, 128), (256, 128), (384, 128), (512, 120)):
            pltpu.sync_copy(acc.at[pl.ds(base + off, sz)],
                            rows.at[0, pl.ds(0, sz)])
            pltpu.sync_copy(rows.at[0, pl.ds(0, sz)],
                            out.at[c, pl.ds(base + off, sz)])

    scratch = []
    if use_table:
        scratch.append(pltpu.VMEM((NCHUNK, CB), jnp.int32))
    scratch += [
        pltpu.VMEM((NCHUNK, CB), jnp.int32),
        pltpu.VMEM((2, CB, dwidth), jnp.float32),
        pltpu.SemaphoreType.DMA,
        pltpu.VMEM_SHARED((ACC_ROWS, dwidth), jnp.float32),
    ]

    return functools.partial(
        pl.kernel,
        out_type=jax.ShapeDtypeStruct((2, ACC_ROWS, dwidth), jnp.float32),
        mesh=_mesh,
        scratch_types=scratch,
        compiler_params=pltpu.CompilerParams(use_tc_tiling_on_sc=False),
    )(body)


DEGW = 8
DH = D // 2
_deg_sc = _make_sc_pass(DEGW, use_table=False)
_agg1_sc = _make_sc_pass(DH, use_table=True)
_agg2_sc = _make_sc_pass(16, use_table=True)


def _lin1_body(x_ref, w_ref, degp_ref, g1a_ref, g1b_ref):
    dp = degp_ref[...]
    dinv = lax.rsqrt(dp[0, :, 0] + dp[1, :, 0])
    h = jnp.dot(x_ref[...], w_ref[...], preferred_element_type=jnp.float32)
    g1 = h * dinv[:, None]
    g1a_ref[...] = g1[:, :DH]
    g1b_ref[...] = g1[:, DH:]


_lin1 = pl.pallas_call(
    _lin1_body,
    grid=(25,),
    in_specs=[
        pl.BlockSpec((400, D), lambda i: (i, 0)),
        pl.BlockSpec((D, D), lambda i: (0, 0)),
        pl.BlockSpec((2, 400, DEGW), lambda i: (0, i, 0)),
    ],
    out_specs=[
        pl.BlockSpec((400, DH), lambda i: (i, 0)),
        pl.BlockSpec((400, DH), lambda i: (i, 0)),
    ],
    out_shape=[
        jax.ShapeDtypeStruct((N, DH), jnp.float32),
        jax.ShapeDtypeStruct((N, DH), jnp.float32),
    ],
)


def _lin2_body(spa_ref, spb_ref, g1a_ref, g1b_ref, degp_ref, b1_ref, w2_ref,
               g2_ref):
    dp = degp_ref[...]
    dinv = lax.rsqrt(dp[0, :, 0] + dp[1, :, 0])
    b1 = b1_ref[...]
    w2 = w2_ref[...]
    spa = spa_ref[...]
    spb = spb_ref[...]
    agga = (spa[0] + spa[1] + g1a_ref[...]) * dinv[:, None] + b1[:, :DH]
    aggb = (spb[0] + spb[1] + g1b_ref[...]) * dinv[:, None] + b1[:, DH:]
    o1a = jnp.maximum(agga, 0.0)
    o1b = jnp.maximum(aggb, 0.0)
    h2 = jnp.sum(o1a * w2[:, :DH], axis=1) + jnp.sum(o1b * w2[:, DH:], axis=1)
    g2 = dinv * h2
    g2_ref[...] = jnp.broadcast_to(g2[:, None], (400, 16))


_lin2 = pl.pallas_call(
    _lin2_body,
    grid=(25,),
    in_specs=[
        pl.BlockSpec((2, 400, DH), lambda i: (0, i, 0)),
        pl.BlockSpec((2, 400, DH), lambda i: (0, i, 0)),
        pl.BlockSpec((400, DH), lambda i: (i, 0)),
        pl.BlockSpec((400, DH), lambda i: (i, 0)),
        pl.BlockSpec((2, 400, DEGW), lambda i: (0, i, 0)),
        pl.BlockSpec((1, D), lambda i: (0, 0)),
        pl.BlockSpec((1, D), lambda i: (0, 0)),
    ],
    out_specs=pl.BlockSpec((400, 16), lambda i: (i, 0)),
    out_shape=jax.ShapeDtypeStruct((N, 16), jnp.float32),
)


def _out_body(s2p_ref, g2_ref, degp_ref, b2_ref, out_ref):
    dp = degp_ref[...]
    dinv = lax.rsqrt(dp[0, :, 0] + dp[1, :, 0])
    sp = s2p_ref[...]
    s2 = sp[0, :, 0] + sp[1, :, 0]
    val = dinv * (s2 + g2_ref[:, 0]) + b2_ref[0, 0]
    out_ref[...] = jax.nn.sigmoid(val)[:, None]


_outk = pl.pallas_call(
    _out_body,
    grid=(25,),
    in_specs=[
        pl.BlockSpec((2, 400, 16), lambda i: (0, i, 0)),
        pl.BlockSpec((400, 16), lambda i: (i, 0)),
        pl.BlockSpec((2, 400, DEGW), lambda i: (0, i, 0)),
        pl.BlockSpec((1, 1), lambda i: (0, 0)),
    ],
    out_specs=pl.BlockSpec((400, 1), lambda i: (i, 0)),
    out_shape=jax.ShapeDtypeStruct((N, 1), jnp.float32),
)


def kernel(x, edge_index, W1, b1, W2, b2):
    src = edge_index[0].astype(jnp.int32)
    dst = edge_index[1].astype(jnp.int32)
    srcp = jnp.concatenate(
        [src.reshape(NW, EPW), jnp.zeros((NW, PAD), jnp.int32)], axis=1
    ).reshape(NW * NCHUNK, CB)
    dstp = jnp.concatenate(
        [dst.reshape(NW, EPW), jnp.full((NW, PAD), N, jnp.int32)], axis=1
    ).reshape(NW * NCHUNK, CB)

    degp = _deg_sc(dstp, jnp.zeros((CB, DEGW), jnp.float32),
                   jnp.ones((CB, DEGW), jnp.float32))
    g1a, g1b = _lin1(x, W1, degp)
    zh = jnp.zeros((CB, DH), jnp.float32)
    s1pa = _agg1_sc(srcp, dstp, g1a, zh)
    s1pb = _agg1_sc(srcp, dstp, g1b, zh)
    g2 = _lin2(s1pa, s1pb, g1a, g1b, degp, b1.reshape(1, D), W2.reshape(1, D))
    s2p = _agg2_sc(srcp, dstp, g2, jnp.zeros((CB, 16), jnp.float32))
    return _outk(s2p, g2, degp, b2.reshape(1, 1))


# speedup vs baseline: 15.8812x; 1.1939x over previous
import functools

import jax
import jax.numpy as jnp
from jax import lax
from jax.experimental import pallas as pl
from jax.experimental.pallas import tpu as pltpu
from jax.experimental.pallas import tpu_sc as plsc

N = 10000
E = 320000
D = 128

NW = 32
EPW = E // NW
CB = 128
NCHUNK = 80
EPW_P = NCHUNK * CB
PAD = EPW_P - EPW
ACC_ROWS = N + 112
ZROWS = ACC_ROWS // 16

_mesh = plsc.VectorSubcoreMesh(core_axis_name="c", subcore_axis_name="s")


def _make_sc_pass(dwidth, use_table, nphase=1):
    def body(*refs):
        if use_table:
            srcp, dstp = refs[0], refs[1]
            tables = refs[2:2 + nphase]
            zrows = refs[2 + nphase]
            outs = refs[3 + nphase:3 + 2 * nphase]
            isrc, idst, rows, sem0, sem1, acc = refs[3 + 2 * nphase:]
        else:
            dstp, zrows, ones, out = refs[:4]
            tables, outs = (None,), (out,)
            idst, rows, sem0, sem1, acc = refs[4:]
        c = lax.axis_index("c")
        s = lax.axis_index("s")
        wid = s * 2 + c
        base = s * ZROWS

        pltpu.sync_copy(zrows, rows.at[2])
        if not use_table:
            pltpu.sync_copy(ones, rows.at[0])
        pltpu.sync_copy(dstp.at[pl.ds(wid * NCHUNK, NCHUNK)], idst)
        if use_table:
            pltpu.sync_copy(srcp.at[pl.ds(wid * NCHUNK, NCHUNK)], isrc)

        for p in range(nphase):
            table, out = tables[p], outs[p]

            for off, sz in ((0, 128), (128, 128), (256, 128), (384, 128),
                            (512, 120)):
                pltpu.sync_copy(rows.at[2, pl.ds(0, sz)],
                                acc.at[pl.ds(base + off, sz)])
            plsc.subcore_barrier()

            if use_table:
                pltpu.async_copy(table.at[isrc.at[0]], rows.at[0], sem0)

                def ebody(k, _):
                    j0 = 2 * k
                    pltpu.async_copy(table.at[isrc.at[j0 + 1]], rows.at[1],
                                     sem1)
                    pltpu.make_async_copy(table.at[isrc.at[j0]], rows.at[0],
                                          sem0).wait()
                    pltpu.sync_copy(rows.at[0], acc.at[idst.at[j0]], add=True)

                    @pl.when(j0 + 2 < NCHUNK)
                    def _():
                        pltpu.async_copy(table.at[isrc.at[j0 + 2]],
                                         rows.at[0], sem0)
                    pltpu.make_async_copy(table.at[isrc.at[j0 + 1]],
                                          rows.at[1], sem1).wait()
                    pltpu.sync_copy(rows.at[1], acc.at[idst.at[j0 + 1]],
                                    add=True)
                    return 0
            else:
                def ebody(k, _):
                    j0 = 2 * k
                    pltpu.sync_copy(rows.at[0], acc.at[idst.at[j0]], add=True)
                    pltpu.sync_copy(rows.at[0], acc.at[idst.at[j0 + 1]],
                                    add=True)
                    return 0
            lax.fori_loop(0, NCHUNK // 2, ebody, 0)
            plsc.subcore_barrier()

            for off, sz in ((0, 128), (128, 128), (256, 128), (384, 128),
                            (512, 120)):
                pltpu.sync_copy(acc.at[pl.ds(base + off, sz)],
                                rows.at[0, pl.ds(0, sz)])
                pltpu.sync_copy(rows.at[0, pl.ds(0, sz)],
                                out.at[c, pl.ds(base + off, sz)])
            if use_table and nphase > 1 and p + 1 < nphase:
                plsc.subcore_barrier()

    scratch = []
    if use_table:
        scratch.append(pltpu.VMEM((NCHUNK, CB), jnp.int32))
    scratch += [
        pltpu.VMEM((NCHUNK, CB), jnp.int32),
        pltpu.VMEM((3, CB, dwidth), jnp.float32),
        pltpu.SemaphoreType.DMA,
        pltpu.SemaphoreType.DMA,
        pltpu.VMEM_SHARED((ACC_ROWS, dwidth), jnp.float32),
    ]

    out_one = jax.ShapeDtypeStruct((2, ACC_ROWS, dwidth), jnp.float32)
    return functools.partial(
        pl.kernel,
        out_type=[out_one] * nphase if nphase > 1 else out_one,
        mesh=_mesh,
        scratch_types=scratch,
        compiler_params=pltpu.CompilerParams(use_tc_tiling_on_sc=False),
    )(body)


DEGW = 8
DH = D // 2
_deg_sc = _make_sc_pass(DEGW, use_table=False)
_agg1_sc = _make_sc_pass(DH, use_table=True, nphase=2)
_agg2_sc = _make_sc_pass(16, use_table=True)


def _lin1_body(x_ref, w_ref, degp_ref, g1a_ref, g1b_ref):
    dp = degp_ref[...]
    dinv = lax.rsqrt(dp[0, :, 0] + dp[1, :, 0])
    h = jnp.dot(x_ref[...], w_ref[...], preferred_element_type=jnp.float32)
    g1 = h * dinv[:, None]
    g1a_ref[...] = g1[:, :DH]
    g1b_ref[...] = g1[:, DH:]


_lin1 = pl.pallas_call(
    _lin1_body,
    grid=(25,),
    in_specs=[
        pl.BlockSpec((400, D), lambda i: (i, 0)),
        pl.BlockSpec((D, D), lambda i: (0, 0)),
        pl.BlockSpec((2, 400, DEGW), lambda i: (0, i, 0)),
    ],
    out_specs=[
        pl.BlockSpec((400, DH), lambda i: (i, 0)),
        pl.BlockSpec((400, DH), lambda i: (i, 0)),
    ],
    out_shape=[
        jax.ShapeDtypeStruct((N, DH), jnp.float32),
        jax.ShapeDtypeStruct((N, DH), jnp.float32),
    ],
)


def _lin2_body(spa_ref, spb_ref, g1a_ref, g1b_ref, degp_ref, b1_ref, w2_ref,
               g2_ref):
    dp = degp_ref[...]
    dinv = lax.rsqrt(dp[0, :, 0] + dp[1, :, 0])
    b1 = b1_ref[...]
    w2 = w2_ref[...]
    spa = spa_ref[...]
    spb = spb_ref[...]
    agga = (spa[0] + spa[1] + g1a_ref[...]) * dinv[:, None] + b1[:, :DH]
    aggb = (spb[0] + spb[1] + g1b_ref[...]) * dinv[:, None] + b1[:, DH:]
    o1a = jnp.maximum(agga, 0.0)
    o1b = jnp.maximum(aggb, 0.0)
    h2 = jnp.sum(o1a * w2[:, :DH], axis=1) + jnp.sum(o1b * w2[:, DH:], axis=1)
    g2 = dinv * h2
    g2_ref[...] = jnp.broadcast_to(g2[:, None], (400, 16))


_lin2 = pl.pallas_call(
    _lin2_body,
    grid=(25,),
    in_specs=[
        pl.BlockSpec((2, 400, DH), lambda i: (0, i, 0)),
        pl.BlockSpec((2, 400, DH), lambda i: (0, i, 0)),
        pl.BlockSpec((400, DH), lambda i: (i, 0)),
        pl.BlockSpec((400, DH), lambda i: (i, 0)),
        pl.BlockSpec((2, 400, DEGW), lambda i: (0, i, 0)),
        pl.BlockSpec((1, D), lambda i: (0, 0)),
        pl.BlockSpec((1, D), lambda i: (0, 0)),
    ],
    out_specs=pl.BlockSpec((400, 16), lambda i: (i, 0)),
    out_shape=jax.ShapeDtypeStruct((N, 16), jnp.float32),
)


def _out_body(s2p_ref, g2_ref, degp_ref, b2_ref, out_ref):
    dp = degp_ref[...]
    dinv = lax.rsqrt(dp[0, :, 0] + dp[1, :, 0])
    sp = s2p_ref[...]
    s2 = sp[0, :, 0] + sp[1, :, 0]
    val = dinv * (s2 + g2_ref[:, 0]) + b2_ref[0, 0]
    out_ref[...] = jax.nn.sigmoid(val)[:, None]


_outk = pl.pallas_call(
    _out_body,
    grid=(25,),
    in_specs=[
        pl.BlockSpec((2, 400, 16), lambda i: (0, i, 0)),
        pl.BlockSpec((400, 16), lambda i: (i, 0)),
        pl.BlockSpec((2, 400, DEGW), lambda i: (0, i, 0)),
        pl.BlockSpec((1, 1), lambda i: (0, 0)),
    ],
    out_specs=pl.BlockSpec((400, 1), lambda i: (i, 0)),
    out_shape=jax.ShapeDtypeStruct((N, 1), jnp.float32),
)


def kernel(x, edge_index, W1, b1, W2, b2):
    src = edge_index[0].astype(jnp.int32)
    dst = edge_index[1].astype(jnp.int32)
    srcp = jnp.concatenate(
        [src.reshape(NW, EPW), jnp.zeros((NW, PAD), jnp.int32)], axis=1
    ).reshape(NW * NCHUNK, CB)
    dstp = jnp.concatenate(
        [dst.reshape(NW, EPW), jnp.full((NW, PAD), N, jnp.int32)], axis=1
    ).reshape(NW * NCHUNK, CB)

    degp = _deg_sc(dstp, jnp.zeros((CB, DEGW), jnp.float32),
                   jnp.ones((CB, DEGW), jnp.float32))
    g1a, g1b = _lin1(x, W1, degp)
    zh = jnp.zeros((CB, DH), jnp.float32)
    s1pa, s1pb = _agg1_sc(srcp, dstp, g1a, g1b, zh)
    g2 = _lin2(s1pa, s1pb, g1a, g1b, degp, b1.reshape(1, D), W2.reshape(1, D))
    s2p = _agg2_sc(srcp, dstp, g2, jnp.zeros((CB, 16), jnp.float32))
    return _outk(s2p, g2, degp, b2.reshape(1, 1))


# speedup vs baseline: 16.0856x; 1.0129x over previous
import functools

import jax
import jax.numpy as jnp
from jax import lax
from jax.experimental import pallas as pl
from jax.experimental.pallas import tpu as pltpu
from jax.experimental.pallas import tpu_sc as plsc

N = 10000
E = 320000
D = 128

NW = 32
EPW = E // NW
CB = 128
NCHUNK = 80
EPW_P = NCHUNK * CB
PAD = EPW_P - EPW
ACC_ROWS = N + 112
ZROWS = ACC_ROWS // 16

_mesh = plsc.VectorSubcoreMesh(core_axis_name="c", subcore_axis_name="s")


def _make_sc_pass(dwidth, use_table, nphase=1):
    def body(*refs):
        if use_table:
            srcp, dstp = refs[0], refs[1]
            tables = refs[2:2 + nphase]
            zrows = refs[2 + nphase]
            outs = refs[3 + nphase:3 + 2 * nphase]
            (isrc, idst, rows, sg0, sg1, sg2, sg3, ss0, ss1, ss2, ss3,
             acc) = refs[3 + 2 * nphase:]
        else:
            dstp, zrows, ones, out = refs[:4]
            tables, outs = (None,), (out,)
            (idst, rows, sg0, sg1, sg2, sg3, ss0, ss1, ss2, ss3,
             acc) = refs[4:]
        sg = (sg0, sg1, sg2, sg3)
        ss = (ss0, ss1, ss2, ss3)
        c = lax.axis_index("c")
        s = lax.axis_index("s")
        wid = s * 2 + c
        base = s * ZROWS

        pltpu.sync_copy(zrows, rows.at[4])
        if not use_table:
            pltpu.sync_copy(ones, rows.at[0])
        pltpu.sync_copy(dstp.at[pl.ds(wid * NCHUNK, NCHUNK)], idst)
        if use_table:
            pltpu.sync_copy(srcp.at[pl.ds(wid * NCHUNK, NCHUNK)], isrc)

        for p in range(nphase):
            table, out = tables[p], outs[p]

            for off, sz in ((0, 128), (128, 128), (256, 128), (384, 128),
                            (512, 120)):
                pltpu.sync_copy(rows.at[4, pl.ds(0, sz)],
                                acc.at[pl.ds(base + off, sz)])
            plsc.subcore_barrier()

            if use_table:
                pltpu.async_copy(table.at[isrc.at[0]], rows.at[0], sg[0])
                pltpu.async_copy(table.at[isrc.at[1]], rows.at[1], sg[1])

                def ebody(k, _):
                    for b in range(4):
                        j = 4 * k + b
                        b2 = (b + 2) % 4
                        pltpu.make_async_copy(table.at[isrc.at[j]],
                                              rows.at[b], sg[b]).wait()
                        pltpu.async_copy(rows.at[b], acc.at[idst.at[j]],
                                         ss[b], add=True)

                        @pl.when(j >= 2)
                        def _():
                            pltpu.make_async_copy(
                                rows.at[b2], acc.at[idst.at[j - 2]],
                                ss[b2]).wait()

                        @pl.when(j + 2 < NCHUNK)
                        def _():
                            pltpu.async_copy(table.at[isrc.at[j + 2]],
                                             rows.at[b2], sg[b2])
                    return 0
                lax.fori_loop(0, NCHUNK // 4, ebody, 0)
                pltpu.make_async_copy(rows.at[2], acc.at[idst.at[NCHUNK - 2]],
                                      ss[2]).wait()
                pltpu.make_async_copy(rows.at[3], acc.at[idst.at[NCHUNK - 1]],
                                      ss[3]).wait()
            else:
                def ebody(k, _):
                    j0 = 2 * k
                    pltpu.sync_copy(rows.at[0], acc.at[idst.at[j0]], add=True)
                    pltpu.sync_copy(rows.at[0], acc.at[idst.at[j0 + 1]],
                                    add=True)
                    return 0
                lax.fori_loop(0, NCHUNK // 2, ebody, 0)
            plsc.subcore_barrier()

            for off, sz in ((0, 128), (128, 128), (256, 128), (384, 128),
                            (512, 120)):
                pltpu.sync_copy(acc.at[pl.ds(base + off, sz)],
                                rows.at[0, pl.ds(0, sz)])
                pltpu.sync_copy(rows.at[0, pl.ds(0, sz)],
                                out.at[c, pl.ds(base + off, sz)])
            if use_table and nphase > 1 and p + 1 < nphase:
                plsc.subcore_barrier()

    scratch = []
    if use_table:
        scratch.append(pltpu.VMEM((NCHUNK, CB), jnp.int32))
    scratch += [
        pltpu.VMEM((NCHUNK, CB), jnp.int32),
        pltpu.VMEM((5, CB, dwidth), jnp.float32),
    ] + [pltpu.SemaphoreType.DMA] * 8 + [
        pltpu.VMEM_SHARED((ACC_ROWS, dwidth), jnp.float32),
    ]

    out_one = jax.ShapeDtypeStruct((2, ACC_ROWS, dwidth), jnp.float32)
    return functools.partial(
        pl.kernel,
        out_type=[out_one] * nphase if nphase > 1 else out_one,
        mesh=_mesh,
        scratch_types=scratch,
        compiler_params=pltpu.CompilerParams(use_tc_tiling_on_sc=False),
    )(body)


DEGW = 8
DH = D // 2
_deg_sc = _make_sc_pass(DEGW, use_table=False)
_agg1_sc = _make_sc_pass(DH, use_table=True, nphase=2)
_agg2_sc = _make_sc_pass(16, use_table=True)


def _lin1_body(x_ref, w_ref, degp_ref, g1a_ref, g1b_ref):
    dp = degp_ref[...]
    dinv = lax.rsqrt(dp[0, :, 0] + dp[1, :, 0])
    h = jnp.dot(x_ref[...], w_ref[...], preferred_element_type=jnp.float32)
    g1 = h * dinv[:, None]
    g1a_ref[...] = g1[:, :DH]
    g1b_ref[...] = g1[:, DH:]


_lin1 = pl.pallas_call(
    _lin1_body,
    grid=(25,),
    in_specs=[
        pl.BlockSpec((400, D), lambda i: (i, 0)),
        pl.BlockSpec((D, D), lambda i: (0, 0)),
        pl.BlockSpec((2, 400, DEGW), lambda i: (0, i, 0)),
    ],
    out_specs=[
        pl.BlockSpec((400, DH), lambda i: (i, 0)),
        pl.BlockSpec((400, DH), lambda i: (i, 0)),
    ],
    out_shape=[
        jax.ShapeDtypeStruct((N, DH), jnp.float32),
        jax.ShapeDtypeStruct((N, DH), jnp.float32),
    ],
)


def _lin2_body(spa_ref, spb_ref, g1a_ref, g1b_ref, degp_ref, b1_ref, w2_ref,
               g2_ref):
    dp = degp_ref[...]
    dinv = lax.rsqrt(dp[0, :, 0] + dp[1, :, 0])
    b1 = b1_ref[...]
    w2 = w2_ref[...]
    spa = spa_ref[...]
    spb = spb_ref[...]
    agga = (spa[0] + spa[1] + g1a_ref[...]) * dinv[:, None] + b1[:, :DH]
    aggb = (spb[0] + spb[1] + g1b_ref[...]) * dinv[:, None] + b1[:, DH:]
    o1a = jnp.maximum(agga, 0.0)
    o1b = jnp.maximum(aggb, 0.0)
    h2 = jnp.sum(o1a * w2[:, :DH], axis=1) + jnp.sum(o1b * w2[:, DH:], axis=1)
    g2 = dinv * h2
    g2_ref[...] = jnp.broadcast_to(g2[:, None], (400, 16))


_lin2 = pl.pallas_call(
    _lin2_body,
    grid=(25,),
    in_specs=[
        pl.BlockSpec((2, 400, DH), lambda i: (0, i, 0)),
        pl.BlockSpec((2, 400, DH), lambda i: (0, i, 0)),
        pl.BlockSpec((400, DH), lambda i: (i, 0)),
        pl.BlockSpec((400, DH), lambda i: (i, 0)),
        pl.BlockSpec((2, 400, DEGW), lambda i: (0, i, 0)),
        pl.BlockSpec((1, D), lambda i: (0, 0)),
        pl.BlockSpec((1, D), lambda i: (0, 0)),
    ],
    out_specs=pl.BlockSpec((400, 16), lambda i: (i, 0)),
    out_shape=jax.ShapeDtypeStruct((N, 16), jnp.float32),
)


def _out_body(s2p_ref, g2_ref, degp_ref, b2_ref, out_ref):
    dp = degp_ref[...]
    dinv = lax.rsqrt(dp[0, :, 0] + dp[1, :, 0])
    sp = s2p_ref[...]
    s2 = sp[0, :, 0] + sp[1, :, 0]
    val = dinv * (s2 + g2_ref[:, 0]) + b2_ref[0, 0]
    out_ref[...] = jax.nn.sigmoid(val)[:, None]


_outk = pl.pallas_call(
    _out_body,
    grid=(25,),
    in_specs=[
        pl.BlockSpec((2, 400, 16), lambda i: (0, i, 0)),
        pl.BlockSpec((400, 16), lambda i: (i, 0)),
        pl.BlockSpec((2, 400, DEGW), lambda i: (0, i, 0)),
        pl.BlockSpec((1, 1), lambda i: (0, 0)),
    ],
    out_specs=pl.BlockSpec((400, 1), lambda i: (i, 0)),
    out_shape=jax.ShapeDtypeStruct((N, 1), jnp.float32),
)


def kernel(x, edge_index, W1, b1, W2, b2):
    src = edge_index[0].astype(jnp.int32)
    dst = edge_index[1].astype(jnp.int32)
    srcp = jnp.concatenate(
        [src.reshape(NW, EPW), jnp.zeros((NW, PAD), jnp.int32)], axis=1
    ).reshape(NW * NCHUNK, CB)
    dstp = jnp.concatenate(
        [dst.reshape(NW, EPW), jnp.full((NW, PAD), N, jnp.int32)], axis=1
    ).reshape(NW * NCHUNK, CB)

    degp = _deg_sc(dstp, jnp.zeros((CB, DEGW), jnp.float32),
                   jnp.ones((CB, DEGW), jnp.float32))
    g1a, g1b = _lin1(x, W1, degp)
    zh = jnp.zeros((CB, DH), jnp.float32)
    s1pa, s1pb = _agg1_sc(srcp, dstp, g1a, g1b, zh)
    g2 = _lin2(s1pa, s1pb, g1a, g1b, degp, b1.reshape(1, D), W2.reshape(1, D))
    s2p = _agg2_sc(srcp, dstp, g2, jnp.zeros((CB, 16), jnp.float32))
    return _outk(s2p, g2, degp, b2.reshape(1, 1))
